# baseline (speedup 1.0000x reference)
import functools

import jax
import jax.numpy as jnp
from jax import lax
from jax.experimental import pallas as pl
from jax.experimental.pallas import tpu as pltpu
from jax.experimental.pallas import tpu_sc as plsc

N = 10000
E = 320000
D = 128
NC = 2
NS = 16
NW = NC * NS
EW = E // NW
CH = 80
NWIN = EW // CH
CHC = 128
NFULL = EW // CHC
TAIL = EW - NFULL * CHC
NB = 4
NP = 10240
RPS = NP // NS
BR = 1000


def _mesh():
    return plsc.VectorSubcoreMesh(core_axis_name="c", subcore_axis_name="s")


def _sc_degree(row):
    @functools.partial(
        pl.kernel,
        mesh=_mesh(),
        out_type=jax.ShapeDtypeStruct((NC * NP,), jnp.float32),
        scratch_types=[
            pltpu.VMEM((EW,), jnp.int32),
            pltpu.VMEM((CH,), jnp.float32),
            pltpu.VMEM((RPS,), jnp.float32),
            pltpu.VMEM_SHARED((NP,), jnp.float32),
        ],
    )
    def k(row_hbm, out_hbm, idx_v, ones_v, tb_v, hist_sh):
        c = lax.axis_index("c")
        s = lax.axis_index("s")
        wid = s * NC + c

        def fill_ones(i, carry):
            ones_v[pl.ds(i * 16, 16)] = jnp.ones((16,), jnp.float32)
            return carry

        lax.fori_loop(0, CH // 16, fill_ones, 0)

        def zero_tb(i, carry):
            tb_v[pl.ds(i * 16, 16)] = jnp.zeros((16,), jnp.float32)
            return carry

        lax.fori_loop(0, RPS // 16, zero_tb, 0)
        pltpu.sync_copy(tb_v, hist_sh.at[pl.ds(s * RPS, RPS)])
        pltpu.sync_copy(row_hbm.at[pl.ds(wid * EW, EW)], idx_v)
        plsc.subcore_barrier()

        def body(w, carry):
            pltpu.sync_copy(ones_v, hist_sh.at[idx_v.at[pl.ds(w * CH, CH)]], add=True)
            return carry

        lax.fori_loop(0, NWIN, body, 0)
        plsc.subcore_barrier()

        pltpu.sync_copy(hist_sh.at[pl.ds(s * RPS, RPS)], tb_v)
        pltpu.sync_copy(tb_v, out_hbm.at[pl.ds(c * NP + s * RPS, RPS)])

    return k(row)


def _sc_scatter(G, row, col):
    @functools.partial(
        pl.kernel,
        mesh=_mesh(),
        out_type=jax.ShapeDtypeStruct((NC * NP, D), jnp.float32),
        scratch_types=[
            pltpu.VMEM((EW,), jnp.int32),
            pltpu.VMEM((CHC,), jnp.int32),
            pltpu.VMEM((CHC,), jnp.int32),
            pltpu.VMEM((2, CHC, D), jnp.float32),
            pltpu.VMEM_SHARED((NP, D), jnp.float32),
            pltpu.SemaphoreType.DMA,
            pltpu.SemaphoreType.DMA,
            pltpu.SemaphoreType.DMA,
        ],
    )
    def k(g_hbm, row_hbm, col_hbm, out_hbm, cidx, rref0, rref1, rbuf, acc,
          gsem, rsem, ssem):
        c = lax.axis_index("c")
        s = lax.axis_index("s")
        wid = s * NC + c
        base = wid * EW
        rrefs = (rref0, rref1)

        def gather_desc(w, b):
            return pltpu.make_async_copy(
                g_hbm.at[cidx.at[pl.ds(w * CHC, CHC)]], rbuf.at[b], gsem)

        def ridx_desc(w, u):
            return pltpu.make_async_copy(
                row_hbm.at[pl.ds(base + w * CHC, CHC)], rrefs[u], rsem)

        def zrow(r, carry):
            def zlane(j, carry2):
                rbuf[0, r, pl.ds(j * 16, 16)] = jnp.zeros((16,), jnp.float32)
                return carry2

            lax.fori_loop(0, D // 16, zlane, 0)
            return carry

        lax.fori_loop(0, CHC, zrow, 0)

        def zcopy(t, carry):
            pltpu.sync_copy(rbuf.at[0], acc.at[pl.ds(s * RPS + t * CHC, CHC)])
            return carry

        lax.fori_loop(0, RPS // CHC, zcopy, 0)

        pltpu.sync_copy(col_hbm.at[pl.ds(base, EW)], cidx)
        plsc.subcore_barrier()

        ridx_desc(0, 0).start()
        pltpu.async_copy(
            g_hbm.at[cidx.at[pl.ds(0, CHC)]], rbuf.at[0], gsem)

        def scatter_desc(u):
            return pltpu.make_async_copy(rbuf.at[u], acc.at[rrefs[u]], ssem)

        def body(g, carry):
            for u in (0, 1):
                w = g * 2 + u
                gather_desc(w, u).wait()
                ridx_desc(w, u).wait()
                pltpu.async_copy(rbuf.at[u], acc.at[rrefs[u]], ssem, add=True)

                @pl.when(w + 1 < NFULL)
                def _():
                    @pl.when(w >= 1)
                    def _():
                        scatter_desc(1 - u).wait()

                    pltpu.async_copy(
                        g_hbm.at[cidx.at[pl.ds((w + 1) * CHC, CHC)]],
                        rbuf.at[1 - u], gsem)
                    ridx_desc(w + 1, 1 - u).start()

            return carry

        lax.fori_loop(0, NFULL // 2, body, 0)
        scatter_desc(0).wait()
        scatter_desc(1).wait()
        if TAIL:
            t0 = NFULL * CHC
            pltpu.async_copy(
                g_hbm.at[cidx.at[pl.ds(t0, TAIL)]],
                rbuf.at[0, pl.ds(0, TAIL)], gsem).wait()
            pltpu.sync_copy(
                row_hbm.at[pl.ds(base + t0, TAIL)], rref0.at[pl.ds(0, TAIL)])
            pltpu.sync_copy(
                rbuf.at[0, pl.ds(0, TAIL)],
                acc.at[rref0.at[pl.ds(0, TAIL)]], add=True)
        plsc.subcore_barrier()

        def epil(t, carry):
            pltpu.sync_copy(acc.at[pl.ds(s * RPS + t * CHC, CHC)], rbuf.at[0])
            pltpu.sync_copy(rbuf.at[0], out_hbm.at[pl.ds(c * NP + s * RPS + t * CHC, CHC)])
            return carry

        lax.fori_loop(0, RPS // CHC, epil, 0)

    return k(G, row, col)


def _tc_transform(H, Wm, b2, histT):
    def body(h_ref, w_ref, b_ref, ht_ref, g_ref):
        hl = lax.dot_general(
            h_ref[...], w_ref[...], (((1,), (1,)), ((), ())),
            preferred_element_type=jnp.float32,
        ) + b_ref[...]
        deg = ht_ref[:, 0:1] + ht_ref[:, 1:2] + 1.0
        g_ref[...] = hl * lax.rsqrt(deg)

    return pl.pallas_call(
        body,
        grid=(N // BR,),
        in_specs=[
            pl.BlockSpec((BR, D), lambda k: (k, 0)),
            pl.BlockSpec((D, D), lambda k: (0, 0)),
            pl.BlockSpec((1, D), lambda k: (0, 0)),
            pl.BlockSpec((BR, NC), lambda k: (k, 0)),
        ],
        out_specs=pl.BlockSpec((BR, D), lambda k: (k, 0)),
        out_shape=jax.ShapeDtypeStruct((N, D), jnp.float32),
    )(H, Wm, b2, histT)


def _tc_finish(P, G, histT):
    def body(p_ref, g_ref, ht_ref, o_ref):
        accsum = p_ref[0] + p_ref[1] + g_ref[...]
        deg = ht_ref[:, 0:1] + ht_ref[:, 1:2] + 1.0
        o_ref[...] = jnp.maximum(accsum * lax.rsqrt(deg), 0.0)

    return pl.pallas_call(
        body,
        grid=(N // BR,),
        in_specs=[
            pl.BlockSpec((NC, BR, D), lambda k: (0, k, 0)),
            pl.BlockSpec((BR, D), lambda k: (k, 0)),
            pl.BlockSpec((BR, NC), lambda k: (k, 0)),
        ],
        out_specs=pl.BlockSpec((BR, D), lambda k: (k, 0)),
        out_shape=jax.ShapeDtypeStruct((N, D), jnp.float32),
    )(P, G, histT)


def kernel(H, edge_index, W, b):
    ei = edge_index.astype(jnp.int32)
    row = ei[0]
    col = ei[1]
    hist = _sc_degree(row)
    histT = hist.reshape(NC, NP).T
    G = _tc_transform(H, W, b.reshape(1, D), histT)
    P = _sc_scatter(G, row, col).reshape(NC, NP, D)
    return _tc_finish(P, G, histT)


# speedup vs baseline: 1.1254x; 1.1254x over previous
import functools

import jax
import jax.numpy as jnp
from jax import lax
from jax.experimental import pallas as pl
from jax.experimental.pallas import tpu as pltpu
from jax.experimental.pallas import tpu_sc as plsc

N = 10000
E = 320000
D = 128
NC = 2
NS = 16
NW = NC * NS
EW = E // NW
CH = 80
NWIN = EW // CH
CHC = 128
NFULL = EW // CHC
TAIL = EW - NFULL * CHC
NB = 4
NP = 10240
RPS = NP // NS
BR = 2048


def _mesh():
    return plsc.VectorSubcoreMesh(core_axis_name="c", subcore_axis_name="s")


def _sc_degree(edge):
    @functools.partial(
        pl.kernel,
        mesh=_mesh(),
        out_type=jax.ShapeDtypeStruct((NC * NP,), jnp.float32),
        scratch_types=[
            pltpu.VMEM((EW,), jnp.int32),
            pltpu.VMEM((CHC,), jnp.float32),
            pltpu.VMEM((RPS,), jnp.float32),
            pltpu.VMEM_SHARED((NP,), jnp.float32),
        ],
    )
    def k(edge_hbm, out_hbm, idx_v, ones_v, tb_v, hist_sh):
        c = lax.axis_index("c")
        s = lax.axis_index("s")
        wid = s * NC + c

        def fill_ones(i, carry):
            ones_v[pl.ds(i * 16, 16)] = jnp.ones((16,), jnp.float32)
            return carry

        lax.fori_loop(0, CHC // 16, fill_ones, 0)

        def zero_tb(i, carry):
            tb_v[pl.ds(i * 16, 16)] = jnp.zeros((16,), jnp.float32)
            return carry

        lax.fori_loop(0, RPS // 16, zero_tb, 0)
        pltpu.sync_copy(tb_v, hist_sh.at[pl.ds(s * RPS, RPS)])
        pltpu.sync_copy(edge_hbm.at[pl.ds(wid * EW, EW)], idx_v)
        plsc.subcore_barrier()

        def body(w, carry):
            pltpu.sync_copy(ones_v, hist_sh.at[idx_v.at[pl.ds(w * CHC, CHC)]], add=True)
            return carry

        lax.fori_loop(0, NFULL, body, 0)
        if TAIL:
            pltpu.sync_copy(
                ones_v.at[pl.ds(0, TAIL)],
                hist_sh.at[idx_v.at[pl.ds(NFULL * CHC, TAIL)]], add=True)
        plsc.subcore_barrier()

        pltpu.sync_copy(hist_sh.at[pl.ds(s * RPS, RPS)], tb_v)
        pltpu.sync_copy(tb_v, out_hbm.at[pl.ds(c * NP + s * RPS, RPS)])

    return k(edge)


def _sc_scatter(G, edge):
    @functools.partial(
        pl.kernel,
        mesh=_mesh(),
        out_type=jax.ShapeDtypeStruct((NC * NP, D), jnp.float32),
        scratch_types=[
            pltpu.VMEM((EW,), jnp.int32),
            pltpu.VMEM((CHC,), jnp.int32),
            pltpu.VMEM((CHC,), jnp.int32),
            pltpu.VMEM((2, CHC, D), jnp.float32),
            pltpu.VMEM_SHARED((NP, D), jnp.float32),
            pltpu.SemaphoreType.DMA,
            pltpu.SemaphoreType.DMA,
            pltpu.SemaphoreType.DMA,
        ],
    )
    def k(g_hbm, edge_hbm, out_hbm, cidx, rref0, rref1, rbuf, acc,
          gsem, rsem, ssem):
        c = lax.axis_index("c")
        s = lax.axis_index("s")
        wid = s * NC + c
        base = wid * EW
        rrefs = (rref0, rref1)

        def gather_desc(w, b):
            return pltpu.make_async_copy(
                g_hbm.at[cidx.at[pl.ds(w * CHC, CHC)]], rbuf.at[b], gsem)

        def ridx_desc(w, u):
            return pltpu.make_async_copy(
                edge_hbm.at[pl.ds(base + w * CHC, CHC)], rrefs[u], rsem)

        def zrow(r, carry):
            def zlane(j, carry2):
                rbuf[0, r, pl.ds(j * 16, 16)] = jnp.zeros((16,), jnp.float32)
                return carry2

            lax.fori_loop(0, D // 16, zlane, 0)
            return carry

        lax.fori_loop(0, CHC, zrow, 0)

        def zcopy(t, carry):
            pltpu.sync_copy(rbuf.at[0], acc.at[pl.ds(s * RPS + t * CHC, CHC)])
            return carry

        lax.fori_loop(0, RPS // CHC, zcopy, 0)

        pltpu.sync_copy(edge_hbm.at[pl.ds(E + base, EW)], cidx)
        plsc.subcore_barrier()

        ridx_desc(0, 0).start()
        pltpu.async_copy(
            g_hbm.at[cidx.at[pl.ds(0, CHC)]], rbuf.at[0], gsem)

        def scatter_desc(u):
            return pltpu.make_async_copy(rbuf.at[u], acc.at[rrefs[u]], ssem)

        def body(g, carry):
            for u in (0, 1):
                w = g * 2 + u
                gather_desc(w, u).wait()
                ridx_desc(w, u).wait()
                pltpu.async_copy(rbuf.at[u], acc.at[rrefs[u]], ssem, add=True)

                @pl.when(w + 1 < NFULL)
                def _():
                    @pl.when(w >= 1)
                    def _():
                        scatter_desc(1 - u).wait()

                    pltpu.async_copy(
                        g_hbm.at[cidx.at[pl.ds((w + 1) * CHC, CHC)]],
                        rbuf.at[1 - u], gsem)
                    ridx_desc(w + 1, 1 - u).start()

            return carry

        lax.fori_loop(0, NFULL // 2, body, 0)
        scatter_desc(0).wait()
        scatter_desc(1).wait()
        if TAIL:
            t0 = NFULL * CHC
            pltpu.async_copy(
                g_hbm.at[cidx.at[pl.ds(t0, TAIL)]],
                rbuf.at[0, pl.ds(0, TAIL)], gsem).wait()
            pltpu.sync_copy(
                edge_hbm.at[pl.ds(base + t0, TAIL)], rref0.at[pl.ds(0, TAIL)])
            pltpu.sync_copy(
                rbuf.at[0, pl.ds(0, TAIL)],
                acc.at[rref0.at[pl.ds(0, TAIL)]], add=True)
        plsc.subcore_barrier()

        def epil(t, carry):
            pltpu.sync_copy(acc.at[pl.ds(s * RPS + t * CHC, CHC)], rbuf.at[0])
            pltpu.sync_copy(rbuf.at[0], out_hbm.at[pl.ds(c * NP + s * RPS + t * CHC, CHC)])
            return carry

        lax.fori_loop(0, RPS // CHC, epil, 0)

    return k(G, edge)


def _tc_transform(H, Wm, b2, hist2):
    def body(h_ref, w_ref, b_ref, ht_ref, g_ref):
        hl = lax.dot_general(
            h_ref[...], w_ref[...], (((1,), (1,)), ((), ())),
            preferred_element_type=jnp.float32,
        ) + b_ref[...]
        r0 = pl.multiple_of(pl.program_id(0) * BR, 128)
        deg = ht_ref[0, pl.ds(r0, BR)] + ht_ref[1, pl.ds(r0, BR)] + 1.0
        g_ref[...] = hl * lax.rsqrt(deg)[:, None]

    return pl.pallas_call(
        body,
        grid=(pl.cdiv(N, BR),),
        in_specs=[
            pl.BlockSpec((BR, D), lambda k: (k, 0)),
            pl.BlockSpec((D, D), lambda k: (0, 0)),
            pl.BlockSpec((1, D), lambda k: (0, 0)),
            pl.BlockSpec((NC, NP), lambda k: (0, 0)),
        ],
        out_specs=pl.BlockSpec((BR, D), lambda k: (k, 0)),
        out_shape=jax.ShapeDtypeStruct((N, D), jnp.float32),
    )(H, Wm, b2, hist2)


def _tc_finish(P, G, hist2):
    def body(p_ref, g_ref, ht_ref, o_ref):
        accsum = p_ref[0] + p_ref[1] + g_ref[...]
        r0 = pl.multiple_of(pl.program_id(0) * BR, 128)
        deg = ht_ref[0, pl.ds(r0, BR)] + ht_ref[1, pl.ds(r0, BR)] + 1.0
        o_ref[...] = jnp.maximum(accsum * lax.rsqrt(deg)[:, None], 0.0)

    return pl.pallas_call(
        body,
        grid=(pl.cdiv(N, BR),),
        in_specs=[
            pl.BlockSpec((NC, BR, D), lambda k: (0, k, 0)),
            pl.BlockSpec((BR, D), lambda k: (k, 0)),
            pl.BlockSpec((NC, NP), lambda k: (0, 0)),
        ],
        out_specs=pl.BlockSpec((BR, D), lambda k: (k, 0)),
        out_shape=jax.ShapeDtypeStruct((N, D), jnp.float32),
    )(P, G, hist2)


def kernel(H, edge_index, W, b):
    ei = edge_index
    if ei.dtype != jnp.int32:
        ei = ei.astype(jnp.int32)
    ei = ei.reshape(2 * E)
    hist2 = _sc_degree(ei).reshape(NC, NP)
    G = _tc_transform(H, W, b.reshape(1, D), hist2)
    P = _sc_scatter(G, ei).reshape(NC, NP, D)
    return _tc_finish(P, G, hist2)


# speedup vs baseline: 1.1327x; 1.0065x over previous
import functools

import jax
import jax.numpy as jnp
from jax import lax
from jax.experimental import pallas as pl
from jax.experimental.pallas import tpu as pltpu
from jax.experimental.pallas import tpu_sc as plsc

N = 10000
E = 320000
D = 128
NC = 2
NS = 16
NW = NC * NS
EW = E // NW
CH = 80
NWIN = EW // CH
CHC = 128
NFULL = EW // CHC
TAIL = EW - NFULL * CHC
NB = 4
NP = 10240
RPS = NP // NS
BR = 2048


def _mesh():
    return plsc.VectorSubcoreMesh(core_axis_name="c", subcore_axis_name="s")


def _sc_degree(edge):
    @functools.partial(
        pl.kernel,
        mesh=_mesh(),
        out_type=jax.ShapeDtypeStruct((NC * NP,), jnp.float32),
        scratch_types=[
            pltpu.VMEM((EW,), jnp.int32),
            pltpu.VMEM((CHC,), jnp.float32),
            pltpu.VMEM((RPS,), jnp.float32),
            pltpu.VMEM_SHARED((NP,), jnp.float32),
        ],
    )
    def k(edge_hbm, out_hbm, idx_v, ones_v, tb_v, hist_sh):
        c = lax.axis_index("c")
        s = lax.axis_index("s")
        wid = s * NC + c

        def fill_ones(i, carry):
            ones_v[pl.ds(i * 16, 16)] = jnp.ones((16,), jnp.float32)
            return carry

        lax.fori_loop(0, CHC // 16, fill_ones, 0)

        def zero_tb(i, carry):
            tb_v[pl.ds(i * 16, 16)] = jnp.zeros((16,), jnp.float32)
            return carry

        lax.fori_loop(0, RPS // 16, zero_tb, 0)
        pltpu.sync_copy(tb_v, hist_sh.at[pl.ds(s * RPS, RPS)])
        pltpu.sync_copy(edge_hbm.at[pl.ds(wid * EW, EW)], idx_v)
        plsc.subcore_barrier()

        def body(w, carry):
            pltpu.sync_copy(ones_v, hist_sh.at[idx_v.at[pl.ds(w * CHC, CHC)]], add=True)
            return carry

        lax.fori_loop(0, NFULL, body, 0)
        if TAIL:
            pltpu.sync_copy(
                ones_v.at[pl.ds(0, TAIL)],
                hist_sh.at[idx_v.at[pl.ds(NFULL * CHC, TAIL)]], add=True)
        plsc.subcore_barrier()

        pltpu.sync_copy(hist_sh.at[pl.ds(s * RPS, RPS)], tb_v)
        pltpu.sync_copy(tb_v, out_hbm.at[pl.ds(c * NP + s * RPS, RPS)])

    return k(edge)


def _sc_scatter(G, edge):
    @functools.partial(
        pl.kernel,
        mesh=_mesh(),
        out_type=jax.ShapeDtypeStruct((NC * NP, D), jnp.float32),
        scratch_types=[
            pltpu.VMEM((EW,), jnp.int32),
            pltpu.VMEM((CHC,), jnp.int32),
            pltpu.VMEM((CHC,), jnp.int32),
            pltpu.VMEM((2, CHC, D), jnp.float32),
            pltpu.VMEM_SHARED((NP, D), jnp.float32),
            pltpu.SemaphoreType.DMA,
            pltpu.SemaphoreType.DMA,
            pltpu.SemaphoreType.DMA,
        ],
    )
    def k(g_hbm, edge_hbm, out_hbm, cidx, rref0, rref1, rbuf, acc,
          gsem, rsem, ssem):
        c = lax.axis_index("c")
        s = lax.axis_index("s")
        wid = s * NC + c
        base = wid * EW
        rrefs = (rref0, rref1)

        def gather_desc(w, b):
            return pltpu.make_async_copy(
                g_hbm.at[cidx.at[pl.ds(w * CHC, CHC)]], rbuf.at[b], gsem)

        def ridx_desc(w, u):
            return pltpu.make_async_copy(
                edge_hbm.at[pl.ds(base + w * CHC, CHC)], rrefs[u], rsem)

        def zrow(r, carry):
            def zlane(j, carry2):
                rbuf[0, r, pl.ds(j * 16, 16)] = jnp.zeros((16,), jnp.float32)
                return carry2

            lax.fori_loop(0, D // 16, zlane, 0)
            return carry

        lax.fori_loop(0, CHC, zrow, 0)

        def zcopy(t, carry):
            pltpu.sync_copy(rbuf.at[0], acc.at[pl.ds(s * RPS + t * CHC, CHC)])
            return carry

        lax.fori_loop(0, RPS // CHC, zcopy, 0)

        pltpu.sync_copy(edge_hbm.at[pl.ds(E + base, EW)], cidx)
        plsc.subcore_barrier()

        ridx_desc(0, 0).start()
        pltpu.async_copy(
            g_hbm.at[cidx.at[pl.ds(0, CHC)]], rbuf.at[0], gsem)

        def scatter_desc(u):
            return pltpu.make_async_copy(rbuf.at[u], acc.at[rrefs[u]], ssem)

        def body(g, carry):
            for u in (0, 1):
                w = g * 2 + u
                gather_desc(w, u).wait()
                ridx_desc(w, u).wait()
                pltpu.async_copy(rbuf.at[u], acc.at[rrefs[u]], ssem, add=True)

                @pl.when(w + 1 < NFULL)
                def _():
                    @pl.when(w >= 1)
                    def _():
                        scatter_desc(1 - u).wait()

                    pltpu.async_copy(
                        g_hbm.at[cidx.at[pl.ds((w + 1) * CHC, CHC)]],
                        rbuf.at[1 - u], gsem)
                    ridx_desc(w + 1, 1 - u).start()

            return carry

        lax.fori_loop(0, NFULL // 2, body, 0)
        scatter_desc(0).wait()
        scatter_desc(1).wait()
        if TAIL:
            t0 = NFULL * CHC
            pltpu.async_copy(
                g_hbm.at[cidx.at[pl.ds(t0, TAIL)]],
                rbuf.at[0, pl.ds(0, TAIL)], gsem).wait()
            pltpu.sync_copy(
                edge_hbm.at[pl.ds(base + t0, TAIL)], rref0.at[pl.ds(0, TAIL)])
            pltpu.sync_copy(
                rbuf.at[0, pl.ds(0, TAIL)],
                acc.at[rref0.at[pl.ds(0, TAIL)]], add=True)
        plsc.subcore_barrier()

        def epil(t, carry):
            pltpu.sync_copy(acc.at[pl.ds(s * RPS + t * CHC, CHC)], rbuf.at[0])
            pltpu.sync_copy(rbuf.at[0], out_hbm.at[pl.ds(c * NP + s * RPS + t * CHC, CHC)])
            return carry

        lax.fori_loop(0, RPS // CHC, epil, 0)

    return k(G, edge)


def _tc_matmul(H, Wm, b2):
    def body(h_ref, w_ref, b_ref, o_ref):
        o_ref[...] = lax.dot_general(
            h_ref[...], w_ref[...], (((1,), (1,)), ((), ())),
            preferred_element_type=jnp.float32,
        ) + b_ref[...]

    return pl.pallas_call(
        body,
        grid=(pl.cdiv(N, BR),),
        in_specs=[
            pl.BlockSpec((BR, D), lambda k: (k, 0)),
            pl.BlockSpec((D, D), lambda k: (0, 0)),
            pl.BlockSpec((1, D), lambda k: (0, 0)),
        ],
        out_specs=pl.BlockSpec((BR, D), lambda k: (k, 0)),
        out_shape=jax.ShapeDtypeStruct((N, D), jnp.float32),
    )(H, Wm, b2)


def _tc_scale(Hl, hist):
    def body(hl_ref, ht_ref, g_ref):
        r0 = pl.multiple_of(pl.program_id(0) * BR, 128)
        deg = ht_ref[pl.ds(r0, BR)] + ht_ref[pl.ds(NP + r0, BR)] + 1.0
        g_ref[...] = hl_ref[...] * lax.rsqrt(deg)[:, None]

    return pl.pallas_call(
        body,
        grid=(pl.cdiv(N, BR),),
        in_specs=[
            pl.BlockSpec((BR, D), lambda k: (k, 0)),
            pl.BlockSpec((NC * NP,), lambda k: (0,)),
        ],
        out_specs=pl.BlockSpec((BR, D), lambda k: (k, 0)),
        out_shape=jax.ShapeDtypeStruct((N, D), jnp.float32),
    )(Hl, hist)


def _tc_finish(P, G, hist):
    def body(p0_ref, p1_ref, g_ref, ht_ref, o_ref):
        accsum = p0_ref[...] + p1_ref[...] + g_ref[...]
        r0 = pl.multiple_of(pl.program_id(0) * BR, 128)
        deg = ht_ref[pl.ds(r0, BR)] + ht_ref[pl.ds(NP + r0, BR)] + 1.0
        o_ref[...] = jnp.maximum(accsum * lax.rsqrt(deg)[:, None], 0.0)

    nb = NP // BR
    return pl.pallas_call(
        body,
        grid=(pl.cdiv(N, BR),),
        in_specs=[
            pl.BlockSpec((BR, D), lambda k: (k, 0)),
            pl.BlockSpec((BR, D), lambda k: (k + nb, 0)),
            pl.BlockSpec((BR, D), lambda k: (k, 0)),
            pl.BlockSpec((NC * NP,), lambda k: (0,)),
        ],
        out_specs=pl.BlockSpec((BR, D), lambda k: (k, 0)),
        out_shape=jax.ShapeDtypeStruct((N, D), jnp.float32),
    )(P, P, G, hist)


def kernel(H, edge_index, W, b):
    ei = edge_index
    if ei.dtype != jnp.int32:
        ei = ei.astype(jnp.int32)
    ei = ei.reshape(2 * E)
    Hl = _tc_matmul(H, W, b.reshape(1, D))
    hist = _sc_degree(ei)
    G = _tc_scale(Hl, hist)
    P = _sc_scatter(G, ei)
    return _tc_finish(P, G, hist)


# speedup vs baseline: 1.1532x; 1.0181x over previous
import functools

import jax
import jax.numpy as jnp
from jax import lax
from jax.experimental import pallas as pl
from jax.experimental.pallas import tpu as pltpu
from jax.experimental.pallas import tpu_sc as plsc

N = 10000
E = 320000
D = 128
NC = 2
NS = 16
NW = NC * NS
EW = E // NW
CH = 80
NWIN = EW // CH
CHC = 128
NFULL = EW // CHC
TAIL = EW - NFULL * CHC
NB = 4
NP = 10240
RPS = NP // NS
BR = 2048


def _mesh():
    return plsc.VectorSubcoreMesh(core_axis_name="c", subcore_axis_name="s")


def _sc_degree(edge):
    @functools.partial(
        pl.kernel,
        mesh=_mesh(),
        out_type=jax.ShapeDtypeStruct((NC * NP,), jnp.float32),
        scratch_types=[
            pltpu.VMEM((EW,), jnp.int32),
            pltpu.VMEM((CHC,), jnp.float32),
            pltpu.VMEM((RPS,), jnp.float32),
            pltpu.VMEM_SHARED((NP,), jnp.float32),
        ],
    )
    def k(edge_hbm, out_hbm, idx_v, ones_v, tb_v, hist_sh):
        c = lax.axis_index("c")
        s = lax.axis_index("s")
        wid = s * NC + c

        def fill_ones(i, carry):
            ones_v[pl.ds(i * 16, 16)] = jnp.ones((16,), jnp.float32)
            return carry

        lax.fori_loop(0, CHC // 16, fill_ones, 0)

        def zero_tb(i, carry):
            tb_v[pl.ds(i * 16, 16)] = jnp.zeros((16,), jnp.float32)
            return carry

        lax.fori_loop(0, RPS // 16, zero_tb, 0)
        pltpu.sync_copy(tb_v, hist_sh.at[pl.ds(s * RPS, RPS)])
        pltpu.sync_copy(edge_hbm.at[pl.ds(wid * EW, EW)], idx_v)
        plsc.subcore_barrier()

        def body(w, carry):
            pltpu.sync_copy(ones_v, hist_sh.at[idx_v.at[pl.ds(w * CHC, CHC)]], add=True)
            return carry

        lax.fori_loop(0, NFULL, body, 0)
        if TAIL:
            pltpu.sync_copy(
                ones_v.at[pl.ds(0, TAIL)],
                hist_sh.at[idx_v.at[pl.ds(NFULL * CHC, TAIL)]], add=True)
        plsc.subcore_barrier()

        pltpu.sync_copy(hist_sh.at[pl.ds(s * RPS, RPS)], tb_v)
        pltpu.sync_copy(tb_v, out_hbm.at[pl.ds(c * NP + s * RPS, RPS)])

    return k(edge)


def _sc_scatter(G, edge):
    @functools.partial(
        pl.kernel,
        mesh=_mesh(),
        out_type=jax.ShapeDtypeStruct((NC * NP, D), jnp.float32),
        scratch_types=[
            pltpu.VMEM((EW,), jnp.int32),
            pltpu.VMEM((CHC,), jnp.int32),
            pltpu.VMEM((CHC,), jnp.int32),
            pltpu.VMEM((2, CHC, D), jnp.float32),
            pltpu.VMEM_SHARED((NP, D), jnp.float32),
            pltpu.SemaphoreType.DMA,
            pltpu.SemaphoreType.DMA,
            pltpu.SemaphoreType.DMA,
        ],
    )
    def k(g_hbm, edge_hbm, out_hbm, cidx, rref0, rref1, rbuf, acc,
          gsem, rsem, ssem):
        c = lax.axis_index("c")
        s = lax.axis_index("s")
        wid = s * NC + c
        base = wid * EW
        rrefs = (rref0, rref1)

        def gather_desc(w, b):
            return pltpu.make_async_copy(
                g_hbm.at[cidx.at[pl.ds(w * CHC, CHC)]], rbuf.at[b], gsem)

        def ridx_desc(w, u):
            return pltpu.make_async_copy(
                edge_hbm.at[pl.ds(base + w * CHC, CHC)], rrefs[u], rsem)

        def zrow(r, carry):
            def zlane(j, carry2):
                rbuf[0, r, pl.ds(j * 16, 16)] = jnp.zeros((16,), jnp.float32)
                return carry2

            lax.fori_loop(0, D // 16, zlane, 0)
            return carry

        lax.fori_loop(0, CHC, zrow, 0)

        for t in range(RPS // CHC):
            pltpu.async_copy(
                rbuf.at[0], acc.at[pl.ds(s * RPS + t * CHC, CHC)], ssem)

        pltpu.sync_copy(edge_hbm.at[pl.ds(E + base, EW)], cidx)
        for t in range(RPS // CHC):
            pltpu.make_async_copy(
                rbuf.at[0], acc.at[pl.ds(s * RPS + t * CHC, CHC)], ssem).wait()
        plsc.subcore_barrier()

        ridx_desc(0, 0).start()
        pltpu.async_copy(
            g_hbm.at[cidx.at[pl.ds(0, CHC)]], rbuf.at[0], gsem)

        def scatter_desc(u):
            return pltpu.make_async_copy(rbuf.at[u], acc.at[rrefs[u]], ssem)

        def body(g, carry):
            for u in (0, 1):
                w = g * 2 + u
                gather_desc(w, u).wait()
                ridx_desc(w, u).wait()
                pltpu.async_copy(rbuf.at[u], acc.at[rrefs[u]], ssem, add=True)

                @pl.when(w + 1 < NFULL)
                def _():
                    @pl.when(w >= 1)
                    def _():
                        scatter_desc(1 - u).wait()

                    pltpu.async_copy(
                        g_hbm.at[cidx.at[pl.ds((w + 1) * CHC, CHC)]],
                        rbuf.at[1 - u], gsem)
                    ridx_desc(w + 1, 1 - u).start()

            return carry

        lax.fori_loop(0, NFULL // 2, body, 0)
        scatter_desc(0).wait()
        scatter_desc(1).wait()
        if TAIL:
            t0 = NFULL * CHC
            pltpu.async_copy(
                g_hbm.at[cidx.at[pl.ds(t0, TAIL)]],
                rbuf.at[0, pl.ds(0, TAIL)], gsem).wait()
            pltpu.sync_copy(
                edge_hbm.at[pl.ds(base + t0, TAIL)], rref0.at[pl.ds(0, TAIL)])
            pltpu.sync_copy(
                rbuf.at[0, pl.ds(0, TAIL)],
                acc.at[rref0.at[pl.ds(0, TAIL)]], add=True)
        plsc.subcore_barrier()

        nchunk = RPS // CHC

        def out_desc(t):
            return pltpu.make_async_copy(
                rbuf.at[t % 2],
                out_hbm.at[pl.ds(c * NP + s * RPS + t * CHC, CHC)], ssem)

        pltpu.sync_copy(acc.at[pl.ds(s * RPS, CHC)], rbuf.at[0])
        for t in range(nchunk):
            out_desc(t).start()
            if t + 1 < nchunk:
                if t >= 1:
                    out_desc(t - 1).wait()
                pltpu.sync_copy(
                    acc.at[pl.ds(s * RPS + (t + 1) * CHC, CHC)],
                    rbuf.at[(t + 1) % 2])
        out_desc(nchunk - 2).wait()
        out_desc(nchunk - 1).wait()

    return k(G, edge)


def _tc_matmul(H, Wm, b2):
    def body(h_ref, w_ref, b_ref, o_ref):
        o_ref[...] = lax.dot_general(
            h_ref[...], w_ref[...], (((1,), (1,)), ((), ())),
            preferred_element_type=jnp.float32,
        ) + b_ref[...]

    return pl.pallas_call(
        body,
        grid=(pl.cdiv(N, BR),),
        in_specs=[
            pl.BlockSpec((BR, D), lambda k: (k, 0)),
            pl.BlockSpec((D, D), lambda k: (0, 0)),
            pl.BlockSpec((1, D), lambda k: (0, 0)),
        ],
        out_specs=pl.BlockSpec((BR, D), lambda k: (k, 0)),
        out_shape=jax.ShapeDtypeStruct((N, D), jnp.float32),
    )(H, Wm, b2)


def _tc_scale(Hl, hist):
    def body(hl_ref, ht_ref, g_ref):
        r0 = pl.multiple_of(pl.program_id(0) * BR, 128)
        deg = ht_ref[pl.ds(r0, BR)] + ht_ref[pl.ds(NP + r0, BR)] + 1.0
        g_ref[...] = hl_ref[...] * lax.rsqrt(deg)[:, None]

    return pl.pallas_call(
        body,
        grid=(pl.cdiv(N, BR),),
        in_specs=[
            pl.BlockSpec((BR, D), lambda k: (k, 0)),
            pl.BlockSpec((NC * NP,), lambda k: (0,)),
        ],
        out_specs=pl.BlockSpec((BR, D), lambda k: (k, 0)),
        out_shape=jax.ShapeDtypeStruct((N, D), jnp.float32),
    )(Hl, hist)


def _tc_finish(P, G, hist):
    def body(p0_ref, p1_ref, g_ref, ht_ref, o_ref):
        accsum = p0_ref[...] + p1_ref[...] + g_ref[...]
        r0 = pl.multiple_of(pl.program_id(0) * BR, 128)
        deg = ht_ref[pl.ds(r0, BR)] + ht_ref[pl.ds(NP + r0, BR)] + 1.0
        o_ref[...] = jnp.maximum(accsum * lax.rsqrt(deg)[:, None], 0.0)

    nb = NP // BR
    return pl.pallas_call(
        body,
        grid=(pl.cdiv(N, BR),),
        in_specs=[
            pl.BlockSpec((BR, D), lambda k: (k, 0)),
            pl.BlockSpec((BR, D), lambda k: (k + nb, 0)),
            pl.BlockSpec((BR, D), lambda k: (k, 0)),
            pl.BlockSpec((NC * NP,), lambda k: (0,)),
        ],
        out_specs=pl.BlockSpec((BR, D), lambda k: (k, 0)),
        out_shape=jax.ShapeDtypeStruct((N, D), jnp.float32),
    )(P, P, G, hist)


def kernel(H, edge_index, W, b):
    ei = edge_index
    if ei.dtype != jnp.int32:
        ei = ei.astype(jnp.int32)
    ei = ei.reshape(2 * E)
    Hl = _tc_matmul(H, W, b.reshape(1, D))
    hist = _sc_degree(ei)
    G = _tc_scale(Hl, hist)
    P = _sc_scatter(G, ei)
    return _tc_finish(P, G, hist)


# speedup vs baseline: 1.1553x; 1.0018x over previous
import functools

import jax
import jax.numpy as jnp
from jax import lax
from jax.experimental import pallas as pl
from jax.experimental.pallas import tpu as pltpu
from jax.experimental.pallas import tpu_sc as plsc

N = 10000
E = 320000
D = 128
NC = 2
NS = 16
NW = NC * NS
EW = E // NW
CHC = 128
NFULL = EW // CHC
TAIL = EW - NFULL * CHC
NP = 10240
RPS = NP // NS
BR = 2048


def _mesh():
    return plsc.VectorSubcoreMesh(core_axis_name="c", subcore_axis_name="s")


def _sc_degree(edge):
    @functools.partial(
        pl.kernel,
        mesh=_mesh(),
        out_type=jax.ShapeDtypeStruct((NC * NP,), jnp.float32),
        scratch_types=[
            pltpu.VMEM((EW,), jnp.int32),
            pltpu.VMEM((CHC,), jnp.float32),
            pltpu.VMEM((RPS,), jnp.float32),
            pltpu.VMEM_SHARED((NP,), jnp.float32),
        ],
    )
    def k(edge_hbm, out_hbm, idx_v, ones_v, tb_v, hist_sh):
        c = lax.axis_index("c")
        s = lax.axis_index("s")
        wid = s * NC + c

        def fill_ones(i, carry):
            ones_v[pl.ds(i * 16, 16)] = jnp.ones((16,), jnp.float32)
            return carry

        lax.fori_loop(0, CHC // 16, fill_ones, 0)

        def zero_tb(i, carry):
            tb_v[pl.ds(i * 16, 16)] = jnp.zeros((16,), jnp.float32)
            return carry

        lax.fori_loop(0, RPS // 16, zero_tb, 0)
        pltpu.sync_copy(tb_v, hist_sh.at[pl.ds(s * RPS, RPS)])
        pltpu.sync_copy(edge_hbm.at[pl.ds(wid * EW, EW)], idx_v)
        plsc.subcore_barrier()

        def body(w, carry):
            pltpu.sync_copy(ones_v, hist_sh.at[idx_v.at[pl.ds(w * CHC, CHC)]], add=True)
            return carry

        lax.fori_loop(0, NFULL, body, 0)
        if TAIL:
            pltpu.sync_copy(
                ones_v.at[pl.ds(0, TAIL)],
                hist_sh.at[idx_v.at[pl.ds(NFULL * CHC, TAIL)]], add=True)
        plsc.subcore_barrier()

        pltpu.sync_copy(hist_sh.at[pl.ds(s * RPS, RPS)], tb_v)
        pltpu.sync_copy(tb_v, out_hbm.at[pl.ds(c * NP + s * RPS, RPS)])

    return k(edge)


def _sc_scatter(G, edge):
    @functools.partial(
        pl.kernel,
        mesh=_mesh(),
        out_type=jax.ShapeDtypeStruct((NC * NP, D), jnp.float32),
        scratch_types=[
            pltpu.VMEM((EW,), jnp.int32),
            pltpu.VMEM((CHC,), jnp.int32),
            pltpu.VMEM((CHC,), jnp.int32),
            pltpu.VMEM((2, CHC, D), jnp.float32),
            pltpu.VMEM_SHARED((NP, D), jnp.float32),
            pltpu.SemaphoreType.DMA,
            pltpu.SemaphoreType.DMA,
            pltpu.SemaphoreType.DMA,
        ],
    )
    def k(g_hbm, edge_hbm, out_hbm, cidx, rref0, rref1, rbuf, acc,
          gsem, rsem, ssem):
        c = lax.axis_index("c")
        s = lax.axis_index("s")
        wid = s * NC + c
        base = wid * EW
        rrefs = (rref0, rref1)

        def gather_desc(w, b):
            return pltpu.make_async_copy(
                g_hbm.at[cidx.at[pl.ds(w * CHC, CHC)]], rbuf.at[b], gsem)

        def ridx_desc(w, u):
            return pltpu.make_async_copy(
                edge_hbm.at[pl.ds(base + w * CHC, CHC)], rrefs[u], rsem)

        def zrow(r, carry):
            def zlane(j, carry2):
                rbuf[0, r, pl.ds(j * 16, 16)] = jnp.zeros((16,), jnp.float32)
                return carry2

            lax.fori_loop(0, D // 16, zlane, 0)
            return carry

        lax.fori_loop(0, CHC, zrow, 0)

        for t in range(RPS // CHC):
            pltpu.async_copy(
                rbuf.at[0], acc.at[pl.ds(s * RPS + t * CHC, CHC)], ssem)

        pltpu.sync_copy(edge_hbm.at[pl.ds(E + base, EW)], cidx)
        for t in range(RPS // CHC):
            pltpu.make_async_copy(
                rbuf.at[0], acc.at[pl.ds(s * RPS + t * CHC, CHC)], ssem).wait()
        plsc.subcore_barrier()

        ridx_desc(0, 0).start()
        pltpu.async_copy(
            g_hbm.at[cidx.at[pl.ds(0, CHC)]], rbuf.at[0], gsem)

        def scatter_desc(u):
            return pltpu.make_async_copy(rbuf.at[u], acc.at[rrefs[u]], ssem)

        def body(g, carry):
            for u in (0, 1):
                w = g * 2 + u
                gather_desc(w, u).wait()
                ridx_desc(w, u).wait()
                pltpu.async_copy(rbuf.at[u], acc.at[rrefs[u]], ssem, add=True)

                @pl.when(w + 1 < NFULL)
                def _():
                    @pl.when(w >= 1)
                    def _():
                        scatter_desc(1 - u).wait()

                    pltpu.async_copy(
                        g_hbm.at[cidx.at[pl.ds((w + 1) * CHC, CHC)]],
                        rbuf.at[1 - u], gsem)
                    ridx_desc(w + 1, 1 - u).start()

            return carry

        lax.fori_loop(0, NFULL // 2, body, 0)
        scatter_desc(0).wait()
        scatter_desc(1).wait()
        if TAIL:
            t0 = NFULL * CHC
            pltpu.async_copy(
                g_hbm.at[cidx.at[pl.ds(t0, TAIL)]],
                rbuf.at[0, pl.ds(0, TAIL)], gsem).wait()
            pltpu.sync_copy(
                edge_hbm.at[pl.ds(base + t0, TAIL)], rref0.at[pl.ds(0, TAIL)])
            pltpu.sync_copy(
                rbuf.at[0, pl.ds(0, TAIL)],
                acc.at[rref0.at[pl.ds(0, TAIL)]], add=True)
        plsc.subcore_barrier()

        nchunk = RPS // CHC

        def out_desc(t):
            return pltpu.make_async_copy(
                rbuf.at[t % 2],
                out_hbm.at[pl.ds(c * NP + s * RPS + t * CHC, CHC)], ssem)

        pltpu.sync_copy(acc.at[pl.ds(s * RPS, CHC)], rbuf.at[0])
        for t in range(nchunk):
            out_desc(t).start()
            if t + 1 < nchunk:
                if t >= 1:
                    out_desc(t - 1).wait()
                pltpu.sync_copy(
                    acc.at[pl.ds(s * RPS + (t + 1) * CHC, CHC)],
                    rbuf.at[(t + 1) % 2])
        out_desc(nchunk - 2).wait()
        out_desc(nchunk - 1).wait()

    return k(G, edge)


def _tc_matmul(H, Wm, b2):
    def body(h_ref, w_ref, b_ref, o_ref):
        o_ref[...] = lax.dot_general(
            h_ref[...], w_ref[...], (((1,), (1,)), ((), ())),
            preferred_element_type=jnp.float32,
        ) + b_ref[...]

    return pl.pallas_call(
        body,
        grid=(pl.cdiv(N, BR),),
        in_specs=[
            pl.BlockSpec((BR, D), lambda k: (k, 0)),
            pl.BlockSpec((D, D), lambda k: (0, 0)),
            pl.BlockSpec((1, D), lambda k: (0, 0)),
        ],
        out_specs=pl.BlockSpec((BR, D), lambda k: (k, 0)),
        out_shape=jax.ShapeDtypeStruct((N, D), jnp.float32),
    )(H, Wm, b2)


def _tc_scale(Hl, hist):
    def body(hl_ref, ht_ref, g_ref):
        r0 = pl.multiple_of(pl.program_id(0) * BR, 128)
        deg = ht_ref[pl.ds(r0, BR)] + ht_ref[pl.ds(NP + r0, BR)] + 1.0
        g_ref[...] = hl_ref[...] * lax.rsqrt(deg)[:, None]

    return pl.pallas_call(
        body,
        grid=(pl.cdiv(N, BR),),
        in_specs=[
            pl.BlockSpec((BR, D), lambda k: (k, 0)),
            pl.BlockSpec((NC * NP,), lambda k: (0,)),
        ],
        out_specs=pl.BlockSpec((BR, D), lambda k: (k, 0)),
        out_shape=jax.ShapeDtypeStruct((N, D), jnp.float32),
    )(Hl, hist)


def _tc_finish(P, G, hist):
    def body(p0_ref, p1_ref, g_ref, ht_ref, o_ref):
        accsum = p0_ref[...] + p1_ref[...] + g_ref[...]
        r0 = pl.multiple_of(pl.program_id(0) * BR, 128)
        deg = ht_ref[pl.ds(r0, BR)] + ht_ref[pl.ds(NP + r0, BR)] + 1.0
        o_ref[...] = jnp.maximum(accsum * lax.rsqrt(deg)[:, None], 0.0)

    nb = NP // BR
    return pl.pallas_call(
        body,
        grid=(pl.cdiv(N, BR),),
        in_specs=[
            pl.BlockSpec((BR, D), lambda k: (k, 0)),
            pl.BlockSpec((BR, D), lambda k: (k + nb, 0)),
            pl.BlockSpec((BR, D), lambda k: (k, 0)),
            pl.BlockSpec((NC * NP,), lambda k: (0,)),
        ],
        out_specs=pl.BlockSpec((BR, D), lambda k: (k, 0)),
        out_shape=jax.ShapeDtypeStruct((N, D), jnp.float32),
    )(P, P, G, hist)


def kernel(H, edge_index, W, b):
    ei = edge_index
    if ei.dtype != jnp.int32:
        ei = ei.astype(jnp.int32)
    ei = ei.reshape(2 * E)
    Hl = _tc_matmul(H, W, b.reshape(1, D))
    hist = _sc_degree(ei)
    G = _tc_scale(Hl, hist)
    P = _sc_scatter(G, ei)
    return _tc_finish(P, G, hist)
